# baseline (speedup 1.0000x reference)
import functools

import jax
import jax.numpy as jnp
from jax import lax
from jax.experimental import pallas as pl
from jax.experimental.pallas import tpu as pltpu
from jax.experimental.pallas import tpu_sc as plsc

_NC = 2
_NS = 16
_NW = _NC * _NS
_L = 16


@functools.lru_cache(maxsize=None)
def _build_sc(B, N):
    npw = N // _NW
    n_iters = npw // _L
    mesh = plsc.VectorSubcoreMesh(core_axis_name="c", subcore_axis_name="s")

    @functools.partial(
        pl.kernel,
        out_type=jax.ShapeDtypeStruct((_NW * 2 * _L,), jnp.float32),
        mesh=mesh,
        compiler_params=pltpu.CompilerParams(needs_layout_passes=False),
        scratch_types=[
            pltpu.VMEM((npw * 5,), jnp.float32),
            pltpu.VMEM((npw * 5,), jnp.float32),
            pltpu.VMEM((npw,), jnp.int32),
            pltpu.VMEM((npw * 4,), jnp.float32),
            pltpu.VMEM((npw,), jnp.float32),
            pltpu.VMEM((npw,), jnp.float32),
            pltpu.VMEM((2 * _L,), jnp.float32),
        ],
    )
    def sc_kern(e_hbm, t_hbm, l_hbm, a_hbm, part_hbm,
                ebuf, tbuf, lbuf, abuf, tsbuf, is2buf, pbuf):
        wid = lax.axis_index("s") * _NC + lax.axis_index("c")
        base_n = wid * npw
        iota = lax.iota(jnp.int32, _L)
        iota4 = iota * 4
        iota5 = iota * 5

        pltpu.sync_copy(a_hbm.at[pl.ds(base_n * 4, npw * 4)], abuf)

        def sig_body(i, c):
            idx0 = iota4 + i * (4 * _L)
            x0 = plsc.load_gather(abuf, [idx0])
            x2 = plsc.load_gather(abuf, [idx0 + 2])
            ts = (x2 - x0) + 1.0
            sg = ts * 0.5
            tsbuf[pl.ds(i * _L, _L)] = ts
            is2buf[pl.ds(i * _L, _L)] = 1.0 / (sg * sg)
            return c

        lax.fori_loop(0, n_iters, sig_body, 0)

        def batch_body(b, acc):
            row0 = pl.multiple_of((b * N + base_n) * 5, 5 * _L)
            lab0 = pl.multiple_of(b * N + base_n, npw)
            pltpu.sync_copy(e_hbm.at[pl.ds(row0, npw * 5)], ebuf)
            pltpu.sync_copy(t_hbm.at[pl.ds(row0, npw * 5)], tbuf)
            pltpu.sync_copy(l_hbm.at[pl.ds(lab0, npw)], lbuf)

            def row_body(j, acc2):
                ak, ac = acc2
                r0 = j * _L
                idx = iota5 + r0 * 5
                dxo = plsc.load_gather(ebuf, [idx])
                dyo = plsc.load_gather(ebuf, [idx + 1])
                dlo = plsc.load_gather(ebuf, [idx + 2])
                dso = plsc.load_gather(ebuf, [idx + 3])
                ao = plsc.load_gather(ebuf, [idx + 4])
                dxt = plsc.load_gather(tbuf, [idx])
                dyt = plsc.load_gather(tbuf, [idx + 1])
                dlt = plsc.load_gather(tbuf, [idx + 2])
                dst = plsc.load_gather(tbuf, [idx + 3])
                at_ = plsc.load_gather(tbuf, [idx + 4])
                lbl = lbuf[pl.ds(r0, _L)]
                ts = tsbuf[pl.ds(r0, _L)]
                is2 = is2buf[pl.ds(r0, _L)]

                aa = ao * ao + 1.0
                bb = at_ * at_ + 1.0
                cc = ao * at_ + 1.0
                ss = ao - at_
                inv_a = 1.0 / aa
                inv_ab = inv_a / bb
                elt = jnp.exp(dlt + dlt)
                est = jnp.exp(dst + dst)
                ielo = jnp.exp(-(dlo + dlo))
                ieso = jnp.exp(-(dso + dso))
                t12 = elt * ielo + est * ieso
                t34 = elt * ieso + est * ielo
                trace = (cc * cc * t12 + ss * ss * t34) * inv_ab
                dx = ts * (dxo - dxt)
                dy = ts * (dyo - dyt)
                u = dx + ao * dy
                v = dy - ao * dx
                dist = (u * u * ielo + v * v * ieso) * (inv_a * is2)
                det = (dlo - dlt) + (dso - dst)
                kld = (trace + dist) * 0.5 + det - 1.0
                pos = lbl == 1
                ak = ak + jnp.where(pos, kld, 0.0)
                ac = ac + jnp.where(pos, 1.0, 0.0)
                return (ak, ac)

            return lax.fori_loop(0, n_iters, row_body, acc)

        zero = jnp.zeros((_L,), jnp.float32)
        acc_k, acc_c = lax.fori_loop(0, B, batch_body, (zero, zero))
        pbuf[pl.ds(0, _L)] = acc_k
        pbuf[pl.ds(_L, _L)] = acc_c
        pltpu.sync_copy(pbuf, part_hbm.at[pl.ds(wid * 2 * _L, 2 * _L)])

    return sc_kern


def _finish_body(p_ref, o_ref):
    x = p_ref[...]
    lane = lax.broadcasted_iota(jnp.int32, x.shape, 1)
    is_k = (lane % (2 * _L)) < _L
    sk = jnp.sum(jnp.where(is_k, x, 0.0))
    sc = jnp.sum(jnp.where(is_k, 0.0, x))
    o_ref[0, 0] = sk / sc


def kernel(out_ellipse, labels, ellipse_targets, anchors):
    B, N, _ = out_ellipse.shape
    parts = _build_sc(B, N)(
        out_ellipse.reshape(-1),
        ellipse_targets.reshape(-1),
        labels.reshape(-1),
        anchors.reshape(-1),
    )
    finish = pl.pallas_call(
        _finish_body,
        out_shape=jax.ShapeDtypeStruct((1, 1), jnp.float32),
        out_specs=pl.BlockSpec(memory_space=pltpu.SMEM),
    )
    res = finish(parts.reshape(8, _NW * 2 * _L // 8))
    return res[0, 0]


# speedup vs baseline: 10.3622x; 10.3622x over previous
import functools

import jax
import jax.numpy as jnp
from jax import lax
from jax.experimental import pallas as pl
from jax.experimental.pallas import tpu as pltpu
from jax.experimental.pallas import tpu_sc as plsc

_NC = 2
_NS = 16
_NW = _NC * _NS
_L = 16


@functools.lru_cache(maxsize=None)
def _build_sc(B, N):
    nt = N // 128
    npt = nt // _NW
    n_iters = npt * (128 // _L)
    mesh = plsc.VectorSubcoreMesh(core_axis_name="c", subcore_axis_name="s")

    @functools.partial(
        pl.kernel,
        out_type=jax.ShapeDtypeStruct((_NW * 2 * _L,), jnp.float32),
        mesh=mesh,
        compiler_params=pltpu.CompilerParams(needs_layout_passes=False),
        scratch_types=[
            pltpu.VMEM((5 * npt, 128), jnp.float32),
            pltpu.VMEM((5 * npt, 128), jnp.float32),
            pltpu.VMEM((npt, 128), jnp.int32),
            pltpu.VMEM((npt, 128), jnp.float32),
            pltpu.VMEM((npt, 128), jnp.float32),
            pltpu.VMEM((npt, 128), jnp.float32),
            pltpu.VMEM((npt, 128), jnp.float32),
            pltpu.VMEM((2 * _L,), jnp.float32),
        ],
    )
    def sc_kern(e_hbm, t_hbm, l_hbm, a_hbm, part_hbm,
                ebuf, tbuf, lbuf, x0buf, x2buf, tsbuf, is2buf, pbuf):
        wid = lax.axis_index("s") * _NC + lax.axis_index("c")
        tc0 = wid * npt

        pltpu.sync_copy(a_hbm.at[pl.ds(tc0, npt), 0, :], x0buf)
        pltpu.sync_copy(a_hbm.at[pl.ds(tc0, npt), 2, :], x2buf)

        def sig_body(k, c):
            i = k >> 3
            j = (k & 7) * _L
            x0 = x0buf[i, pl.ds(j, _L)]
            x2 = x2buf[i, pl.ds(j, _L)]
            ts = (x2 - x0) + 1.0
            tsbuf[i, pl.ds(j, _L)] = ts
            is2buf[i, pl.ds(j, _L)] = 4.0 / (ts * ts)
            return c

        lax.fori_loop(0, n_iters, sig_body, 0)

        def batch_body(b, acc):
            tr = b >> 3
            sl = b & 7
            pltpu.sync_copy(e_hbm.at[:, tr, pl.ds(tc0, npt), sl, :], ebuf.reshape(5, npt, 128))
            pltpu.sync_copy(t_hbm.at[:, tr, pl.ds(tc0, npt), sl, :], tbuf.reshape(5, npt, 128))
            pltpu.sync_copy(l_hbm.at[tr, pl.ds(tc0, npt), sl, :], lbuf)

            def row_body(k, acc2):
                ak, ac = acc2
                i = k >> 3
                j = (k & 7) * _L
                dxo = ebuf[i, pl.ds(j, _L)]
                dyo = ebuf[i + npt, pl.ds(j, _L)]
                dlo = ebuf[i + 2 * npt, pl.ds(j, _L)]
                dso = ebuf[i + 3 * npt, pl.ds(j, _L)]
                ao = ebuf[i + 4 * npt, pl.ds(j, _L)]
                dxt = tbuf[i, pl.ds(j, _L)]
                dyt = tbuf[i + npt, pl.ds(j, _L)]
                dlt = tbuf[i + 2 * npt, pl.ds(j, _L)]
                dst = tbuf[i + 3 * npt, pl.ds(j, _L)]
                at_ = tbuf[i + 4 * npt, pl.ds(j, _L)]
                lbl = lbuf[i, pl.ds(j, _L)]
                ts = tsbuf[i, pl.ds(j, _L)]
                is2 = is2buf[i, pl.ds(j, _L)]

                aa = ao * ao + 1.0
                bb = at_ * at_ + 1.0
                cc = ao * at_ + 1.0
                ss = ao - at_
                inv_a = 1.0 / aa
                inv_ab = inv_a / bb
                elt = jnp.exp(dlt + dlt)
                est = jnp.exp(dst + dst)
                ielo = jnp.exp(-(dlo + dlo))
                ieso = jnp.exp(-(dso + dso))
                t12 = elt * ielo + est * ieso
                t34 = elt * ieso + est * ielo
                trace = (cc * cc * t12 + ss * ss * t34) * inv_ab
                dx = ts * (dxo - dxt)
                dy = ts * (dyo - dyt)
                u = dx + ao * dy
                v = dy - ao * dx
                dist = (u * u * ielo + v * v * ieso) * (inv_a * is2)
                det = (dlo - dlt) + (dso - dst)
                kld = (trace + dist) * 0.5 + det - 1.0
                pos = lbl == 1
                ak = ak + jnp.where(pos, kld, 0.0)
                ac = ac + jnp.where(pos, 1.0, 0.0)
                return (ak, ac)

            return lax.fori_loop(0, n_iters, row_body, acc)

        zero = jnp.zeros((_L,), jnp.float32)
        acc_k, acc_c = lax.fori_loop(0, B, batch_body, (zero, zero))
        pbuf[pl.ds(0, _L)] = acc_k
        pbuf[pl.ds(_L, _L)] = acc_c
        pltpu.sync_copy(pbuf, part_hbm.at[pl.ds(wid * 2 * _L, 2 * _L)])

    return sc_kern


def _finish_body(p_ref, o_ref):
    x = p_ref[...]
    lane = lax.broadcasted_iota(jnp.int32, x.shape, 1)
    is_k = (lane % (2 * _L)) < _L
    sk = jnp.sum(jnp.where(is_k, x, 0.0))
    sc = jnp.sum(jnp.where(is_k, 0.0, x))
    o_ref[0, 0] = sk / sc


def kernel(out_ellipse, labels, ellipse_targets, anchors):
    B, N, F = out_ellipse.shape
    nt = N // 128
    nb = B // 8
    e5 = out_ellipse.transpose(2, 0, 1).reshape(F, nb, 8, nt, 128).transpose(0, 1, 3, 2, 4)
    t5 = ellipse_targets.transpose(2, 0, 1).reshape(F, nb, 8, nt, 128).transpose(0, 1, 3, 2, 4)
    l4 = labels.reshape(nb, 8, nt, 128).transpose(0, 2, 1, 3)
    a3 = anchors.transpose(1, 0).reshape(4, nt, 128).transpose(1, 0, 2)
    parts = _build_sc(B, N)(e5, t5, l4, a3)
    finish = pl.pallas_call(
        _finish_body,
        out_shape=jax.ShapeDtypeStruct((1, 1), jnp.float32),
        out_specs=pl.BlockSpec(memory_space=pltpu.SMEM),
    )
    res = finish(parts.reshape(8, _NW * 2 * _L // 8))
    return res[0, 0]


# speedup vs baseline: 10.4307x; 1.0066x over previous
import functools

import jax
import jax.numpy as jnp
from jax import lax
from jax.experimental import pallas as pl
from jax.experimental.pallas import tpu as pltpu
from jax.experimental.pallas import tpu_sc as plsc

_NC = 2
_NS = 16
_NW = _NC * _NS
_L = 16


@functools.lru_cache(maxsize=None)
def _build_sc(B, N):
    nt = N // 128
    npt = nt // _NW
    mesh = plsc.VectorSubcoreMesh(core_axis_name="c", subcore_axis_name="s")

    @functools.partial(
        pl.kernel,
        out_type=jax.ShapeDtypeStruct((_NW * 2 * _L,), jnp.float32),
        mesh=mesh,
        compiler_params=pltpu.CompilerParams(needs_layout_passes=False),
        scratch_types=[
            pltpu.VMEM((5 * npt, 128), jnp.float32),
            pltpu.VMEM((5 * npt, 128), jnp.float32),
            pltpu.VMEM((5 * npt, 128), jnp.float32),
            pltpu.VMEM((5 * npt, 128), jnp.float32),
            pltpu.VMEM((npt, 128), jnp.int32),
            pltpu.VMEM((npt, 128), jnp.int32),
            pltpu.VMEM((npt, 128), jnp.float32),
            pltpu.VMEM((npt, 128), jnp.float32),
            pltpu.VMEM((npt, 128), jnp.float32),
            pltpu.VMEM((npt, 128), jnp.float32),
            pltpu.VMEM((2 * _L,), jnp.float32),
            pltpu.SemaphoreType.DMA,
            pltpu.SemaphoreType.DMA,
        ],
    )
    def sc_kern(e_hbm, t_hbm, l_hbm, a_hbm, part_hbm,
                ebuf0, ebuf1, tbuf0, tbuf1, lbuf0, lbuf1,
                x0buf, x2buf, tsbuf, is2buf, pbuf, sem0, sem1):
        wid = lax.axis_index("s") * _NC + lax.axis_index("c")
        tc0 = wid * npt

        pltpu.sync_copy(a_hbm.at[pl.ds(tc0, npt), 0, :], x0buf)
        pltpu.sync_copy(a_hbm.at[pl.ds(tc0, npt), 2, :], x2buf)

        def sig_body(k, c):
            i = k >> 3
            j = (k & 7) * _L
            x0 = x0buf[i, pl.ds(j, _L)]
            x2 = x2buf[i, pl.ds(j, _L)]
            ts = (x2 - x0) + 1.0
            tsbuf[i, pl.ds(j, _L)] = ts
            is2buf[i, pl.ds(j, _L)] = 4.0 / (ts * ts)
            return c

        lax.fori_loop(0, npt * 8, sig_body, 0)

        def issue(b, eb, tb, lb, sem):
            tr = b >> 3
            sl = b & 7
            pltpu.async_copy(
                e_hbm.at[:, tr, pl.ds(tc0, npt), sl, :], eb.reshape(5, npt, 128), sem)
            pltpu.async_copy(
                t_hbm.at[:, tr, pl.ds(tc0, npt), sl, :], tb.reshape(5, npt, 128), sem)
            pltpu.async_copy(l_hbm.at[tr, pl.ds(tc0, npt), sl, :], lb, sem)

        def drain(eb, tb, lb, sem):
            pltpu.make_async_copy(
                e_hbm.at[:, 0, pl.ds(0, npt), 0, :], eb.reshape(5, npt, 128), sem).wait()
            pltpu.make_async_copy(
                t_hbm.at[:, 0, pl.ds(0, npt), 0, :], tb.reshape(5, npt, 128), sem).wait()
            pltpu.make_async_copy(l_hbm.at[0, pl.ds(0, npt), 0, :], lb, sem).wait()

        def compute(eb, tb, lb, acc):
            def tile_body(i, acc2):
                ak, ac = acc2
                for j8 in range(8):
                    j = j8 * _L
                    dxo = eb[i, pl.ds(j, _L)]
                    dyo = eb[i + npt, pl.ds(j, _L)]
                    dlo = eb[i + 2 * npt, pl.ds(j, _L)]
                    dso = eb[i + 3 * npt, pl.ds(j, _L)]
                    ao = eb[i + 4 * npt, pl.ds(j, _L)]
                    dxt = tb[i, pl.ds(j, _L)]
                    dyt = tb[i + npt, pl.ds(j, _L)]
                    dlt = tb[i + 2 * npt, pl.ds(j, _L)]
                    dst = tb[i + 3 * npt, pl.ds(j, _L)]
                    at_ = tb[i + 4 * npt, pl.ds(j, _L)]
                    lbl = lb[i, pl.ds(j, _L)]
                    ts = tsbuf[i, pl.ds(j, _L)]
                    is2 = is2buf[i, pl.ds(j, _L)]

                    aa = ao * ao + 1.0
                    bb = at_ * at_ + 1.0
                    cc = ao * at_ + 1.0
                    ss = ao - at_
                    elt = jnp.exp(dlt + dlt)
                    est = jnp.exp(dst + dst)
                    ielo = jnp.exp(-(dlo + dlo))
                    ieso = jnp.exp(-(dso + dso))
                    t12 = elt * ielo + est * ieso
                    t34 = elt * ieso + est * ielo
                    tn = cc * cc * t12 + ss * ss * t34
                    dx = ts * (dxo - dxt)
                    dy = ts * (dyo - dyt)
                    u = dx + ao * dy
                    v = dy - ao * dx
                    dn = (u * u * ielo + v * v * ieso) * is2
                    qh = 0.5 / (aa * bb)
                    det = (dlo - dlt) + (dso - dst)
                    kld = (tn + bb * dn) * qh + det - 1.0
                    pos = lbl == 1
                    ak = ak + jnp.where(pos, kld, 0.0)
                    ac = ac + jnp.where(pos, 1.0, 0.0)
                return (ak, ac)

            return lax.fori_loop(0, npt, tile_body, acc)

        zero = jnp.zeros((_L,), jnp.float32)
        issue(0, ebuf0, tbuf0, lbuf0, sem0)

        def g_body(g, acc):
            b0 = g * 2
            issue(b0 + 1, ebuf1, tbuf1, lbuf1, sem1)
            drain(ebuf0, tbuf0, lbuf0, sem0)
            acc = compute(ebuf0, tbuf0, lbuf0, acc)

            @pl.when(b0 + 2 < B)
            def _():
                issue(b0 + 2, ebuf0, tbuf0, lbuf0, sem0)

            drain(ebuf1, tbuf1, lbuf1, sem1)
            return compute(ebuf1, tbuf1, lbuf1, acc)

        acc_k, acc_c = lax.fori_loop(0, B // 2, g_body, (zero, zero))
        pbuf[pl.ds(0, _L)] = acc_k
        pbuf[pl.ds(_L, _L)] = acc_c
        pltpu.sync_copy(pbuf, part_hbm.at[pl.ds(wid * 2 * _L, 2 * _L)])

    return sc_kern


def _finish_body(p_ref, o_ref):
    x = p_ref[...]
    lane = lax.broadcasted_iota(jnp.int32, x.shape, 1)
    is_k = (lane % (2 * _L)) < _L
    sk = jnp.sum(jnp.where(is_k, x, 0.0))
    sc = jnp.sum(jnp.where(is_k, 0.0, x))
    o_ref[0, 0] = sk / sc


def kernel(out_ellipse, labels, ellipse_targets, anchors):
    B, N, F = out_ellipse.shape
    nt = N // 128
    nb = B // 8
    e5 = out_ellipse.transpose(2, 0, 1).reshape(F, nb, 8, nt, 128).transpose(0, 1, 3, 2, 4)
    t5 = ellipse_targets.transpose(2, 0, 1).reshape(F, nb, 8, nt, 128).transpose(0, 1, 3, 2, 4)
    l4 = labels.reshape(nb, 8, nt, 128).transpose(0, 2, 1, 3)
    a3 = anchors.transpose(1, 0).reshape(4, nt, 128).transpose(1, 0, 2)
    parts = _build_sc(B, N)(e5, t5, l4, a3)
    finish = pl.pallas_call(
        _finish_body,
        out_shape=jax.ShapeDtypeStruct((1, 1), jnp.float32),
        out_specs=pl.BlockSpec(memory_space=pltpu.SMEM),
    )
    res = finish(parts.reshape(8, _NW * 2 * _L // 8))
    return res[0, 0]


# speedup vs baseline: 22.7818x; 2.1841x over previous
import functools

import jax
import jax.numpy as jnp
from jax import lax
from jax.experimental import pallas as pl
from jax.experimental.pallas import tpu as pltpu
from jax.experimental.pallas import tpu_sc as plsc

_NC = 2
_NS = 16
_NW = _NC * _NS
_L = 16


@functools.lru_cache(maxsize=None)
def _build_sc(B, N):
    nt = N // 128
    npt = nt // _NW
    mesh = plsc.VectorSubcoreMesh(core_axis_name="c", subcore_axis_name="s")

    @functools.partial(
        pl.kernel,
        out_type=jax.ShapeDtypeStruct((_NW * 2 * _L,), jnp.float32),
        mesh=mesh,
        compiler_params=pltpu.CompilerParams(needs_layout_passes=False),
        scratch_types=[
            pltpu.VMEM((5 * npt, 128), jnp.float32),
            pltpu.VMEM((5 * npt, 128), jnp.float32),
            pltpu.VMEM((5 * npt, 128), jnp.float32),
            pltpu.VMEM((5 * npt, 128), jnp.float32),
            pltpu.VMEM((npt, 128), jnp.int32),
            pltpu.VMEM((npt, 128), jnp.int32),
            pltpu.VMEM((2 * _L,), jnp.float32),
            pltpu.SemaphoreType.DMA,
            pltpu.SemaphoreType.DMA,
        ],
    )
    def sc_kern(e_hbm, t_hbm, l_hbm, part_hbm,
                ebuf0, ebuf1, tbuf0, tbuf1, lbuf0, lbuf1,
                pbuf, sem0, sem1):
        wid = lax.axis_index("s") * _NC + lax.axis_index("c")
        tc0 = wid * npt

        def issue(b, eb, tb, lb, sem):
            tr = b >> 3
            sl = b & 7
            pltpu.async_copy(
                e_hbm.at[:, tr, pl.ds(tc0, npt), sl, :], eb.reshape(5, npt, 128), sem)
            pltpu.async_copy(
                t_hbm.at[:, tr, pl.ds(tc0, npt), sl, :], tb.reshape(5, npt, 128), sem)
            pltpu.async_copy(l_hbm.at[tr, pl.ds(tc0, npt), sl, :], lb, sem)

        def drain(eb, tb, lb, sem):
            pltpu.make_async_copy(
                e_hbm.at[:, 0, pl.ds(0, npt), 0, :], eb.reshape(5, npt, 128), sem).wait()
            pltpu.make_async_copy(
                t_hbm.at[:, 0, pl.ds(0, npt), 0, :], tb.reshape(5, npt, 128), sem).wait()
            pltpu.make_async_copy(l_hbm.at[0, pl.ds(0, npt), 0, :], lb, sem).wait()

        def compute(eb, tb, lb, acc):
            def tile_body(k, acc2):
                ak, ac = acc2
                i = k >> 1
                jb = (k & 1) * (4 * _L)
                for j4 in range(4):
                    j = jb + j4 * _L
                    dxo = eb[i, pl.ds(j, _L)]
                    dyo = eb[i + npt, pl.ds(j, _L)]
                    dlo = eb[i + 2 * npt, pl.ds(j, _L)]
                    dso = eb[i + 3 * npt, pl.ds(j, _L)]
                    ao = eb[i + 4 * npt, pl.ds(j, _L)]
                    dxt = tb[i, pl.ds(j, _L)]
                    dyt = tb[i + npt, pl.ds(j, _L)]
                    dlt = tb[i + 2 * npt, pl.ds(j, _L)]
                    dst = tb[i + 3 * npt, pl.ds(j, _L)]
                    at_ = tb[i + 4 * npt, pl.ds(j, _L)]
                    lbl = lb[i, pl.ds(j, _L)]

                    aa = ao * ao + 1.0
                    bb = at_ * at_ + 1.0
                    cc = ao * at_ + 1.0
                    ss = ao - at_
                    elt = jnp.exp(dlt + dlt)
                    est = jnp.exp(dst + dst)
                    ielo = jnp.exp(-(dlo + dlo))
                    ieso = jnp.exp(-(dso + dso))
                    t12 = elt * ielo + est * ieso
                    t34 = elt * ieso + est * ielo
                    tn = cc * cc * t12 + ss * ss * t34
                    dx = dxo - dxt
                    dy = dyo - dyt
                    u = dx + ao * dy
                    v = dy - ao * dx
                    dn = u * u * ielo + v * v * ieso
                    qh = 0.5 / (aa * bb)
                    det = (dlo - dlt) + (dso - dst)
                    kld = (tn + (4.0 * bb) * dn) * qh + det
                    lf = lbl.astype(jnp.float32)
                    ak = ak + kld * lf
                    ac = ac + lf
                return (ak, ac)

            return lax.fori_loop(0, npt * 2, tile_body, acc)

        zero = jnp.zeros((_L,), jnp.float32)
        issue(0, ebuf0, tbuf0, lbuf0, sem0)

        def g_body(g, acc):
            b0 = g * 2
            issue(b0 + 1, ebuf1, tbuf1, lbuf1, sem1)
            drain(ebuf0, tbuf0, lbuf0, sem0)
            acc = compute(ebuf0, tbuf0, lbuf0, acc)

            @pl.when(b0 + 2 < B)
            def _():
                issue(b0 + 2, ebuf0, tbuf0, lbuf0, sem0)

            drain(ebuf1, tbuf1, lbuf1, sem1)
            return compute(ebuf1, tbuf1, lbuf1, acc)

        acc_k, acc_c = lax.fori_loop(0, B // 2, g_body, (zero, zero))
        pbuf[pl.ds(0, _L)] = acc_k
        pbuf[pl.ds(_L, _L)] = acc_c
        pltpu.sync_copy(pbuf, part_hbm.at[pl.ds(wid * 2 * _L, 2 * _L)])

    return sc_kern


def _finish_body(p_ref, o_ref):
    x = p_ref[...]
    lane = lax.broadcasted_iota(jnp.int32, x.shape, 1)
    is_k = (lane % (2 * _L)) < _L
    sk = jnp.sum(jnp.where(is_k, x, 0.0))
    sc = jnp.sum(jnp.where(is_k, 0.0, x))
    o_ref[0, 0] = sk / sc - 1.0


def kernel(out_ellipse, labels, ellipse_targets, anchors):
    B, N, F = out_ellipse.shape
    nt = N // 128
    nb = B // 8
    e5 = out_ellipse.transpose(2, 0, 1).reshape(F, nb, 8, nt, 128).transpose(0, 1, 3, 2, 4)
    t5 = ellipse_targets.transpose(2, 0, 1).reshape(F, nb, 8, nt, 128).transpose(0, 1, 3, 2, 4)
    l4 = labels.reshape(nb, 8, nt, 128).transpose(0, 2, 1, 3)
    parts = _build_sc(B, N)(e5, t5, l4)
    finish = pl.pallas_call(
        _finish_body,
        out_shape=jax.ShapeDtypeStruct((1, 1), jnp.float32),
        out_specs=pl.BlockSpec(memory_space=pltpu.SMEM),
    )
    res = finish(parts.reshape(8, _NW * 2 * _L // 8))
    return res[0, 0]
